# baseline (speedup 1.0000x reference)
import functools

import jax
import jax.numpy as jnp
from jax import lax
from jax.experimental import pallas as pl
from jax.experimental.pallas import tpu as pltpu
from jax.experimental.pallas import tpu_sc as plsc

N = 10000
NP = 10240
E = 320000
D = 128
HD = D // 2
G = 128
C = 10

NUM_WORKERS = 32
EW = 64
EP = 327680
NBUF = 8
TRASH = NP - 1
ROWS_PER_TILE = NP // 16
TCH = EP // (16 * EW)
PARTS = 8
PCH = TCH // PARTS
DCH = EP // (NUM_WORKERS * EW)
RB = 2048
TC_GRID = NP // RB

@functools.cache
def _sc_degree_kernel():
    mesh = plsc.VectorSubcoreMesh(core_axis_name="c", subcore_axis_name="s")
    return functools.partial(
        pl.kernel,
        mesh=mesh,
        out_type=jax.ShapeDtypeStruct((NUM_WORKERS * NP,), jnp.float32),
        scratch_types=[
            pltpu.VMEM((DCH, EW), jnp.int32),
            pltpu.VMEM((NP,), jnp.float32),
        ],
        compiler_params=pltpu.CompilerParams(needs_layout_passes=False),
    )(_sc_degree_body)


def _sc_degree_body(dst_hbm, out_hbm, dst_v, deg_v):
    cid = lax.axis_index("c")
    sid = lax.axis_index("s")
    w = cid * 16 + sid
    pltpu.sync_copy(dst_hbm.at[pl.ds(w * DCH, DCH)], dst_v)

    zeros16 = jnp.zeros((16,), jnp.float32)

    def _zero(i, carry):
        deg_v[pl.ds(i * 16, 16)] = zeros16
        return carry

    lax.fori_loop(0, NP // 16, _zero, 0)

    ones16 = jnp.ones((16,), jnp.float32)

    def _count(j, carry):
        for k in range(EW // 16):
            idx = dst_v[j, pl.ds(k * 16, 16)]
            plsc.addupdate_scatter(deg_v, [idx], ones16)
        return carry

    lax.fori_loop(0, DCH, _count, 0)
    pltpu.sync_copy(deg_v, out_hbm.at[pl.ds(w * NP, NP)])


@functools.cache
def _sc_scatter_kernel():
    mesh = plsc.VectorSubcoreMesh(core_axis_name="c", subcore_axis_name="s")
    return functools.partial(
        pl.kernel,
        mesh=mesh,
        out_type=(
            jax.ShapeDtypeStruct((NP, HD), jnp.float32),
            jax.ShapeDtypeStruct((NP, HD), jnp.float32),
        ),
        scratch_types=[
            pltpu.VMEM((PCH, EW), jnp.int32),
            pltpu.VMEM((PCH, EW), jnp.int32),
        ]
        + [pltpu.VMEM((EW, HD), jnp.float32) for _ in range(NBUF)]
        + [
            pltpu.VMEM_SHARED((NP, HD), jnp.float32),
            pltpu.VMEM_SHARED((NP, HD), jnp.float32),
        ]
        + [pltpu.SemaphoreType.DMA for _ in range(NBUF)],
        compiler_params=pltpu.CompilerParams(needs_layout_passes=False,
                                             use_tc_tiling_on_sc=False),
    )(_sc_scatter_body)


def _sc_scatter_body(src_hbm, dst_hbm, ylo_hbm, yhi_hbm, out_lo, out_hi,
                     src_v, dst_v, *rest):
    rows_bufs = rest[:NBUF]
    y_sh = rest[NBUF]
    acc_sh = rest[NBUF + 1]
    sems = rest[NBUF + 2:]
    rows0 = rows_bufs[0]
    cid = lax.axis_index("c")
    sid = lax.axis_index("s")
    row0 = sid * ROWS_PER_TILE

    @pl.when(cid == 0)
    def _():
        pltpu.sync_copy(ylo_hbm.at[pl.ds(row0, ROWS_PER_TILE)],
                        y_sh.at[pl.ds(row0, ROWS_PER_TILE)])

    @pl.when(cid == 1)
    def _():
        pltpu.sync_copy(yhi_hbm.at[pl.ds(row0, ROWS_PER_TILE)],
                        y_sh.at[pl.ds(row0, ROWS_PER_TILE)])

    zeros16 = jnp.zeros((16,), jnp.float32)

    def _zero(i, carry):
        for k in range(HD // 16):
            rows0[i, pl.ds(k * 16, 16)] = zeros16
        return carry

    lax.fori_loop(0, EW, _zero, 0)

    def _zero_acc(k, carry):
        pltpu.sync_copy(rows0, acc_sh.at[pl.ds(row0 + k * EW, EW)])
        return carry

    lax.fori_loop(0, ROWS_PER_TILE // EW, _zero_acc, 0)
    plsc.subcore_barrier()

    bufs = tuple(zip(rows_bufs, sems))
    for h in range(PARTS):
        base = sid * TCH + h * PCH
        pltpu.sync_copy(src_hbm.at[pl.ds(base, PCH)], src_v)
        pltpu.sync_copy(dst_hbm.at[pl.ds(base, PCH)], dst_v)
        for p in range(NBUF - 1):
            pltpu.make_async_copy(y_sh.at[src_v.at[p]], rows_bufs[p],
                                  sems[p]).start()

        def _edge_group(jj, carry):
            for b, (rows, sem) in enumerate(bufs):
                c = jj * NBUF + b
                pltpu.make_async_copy(y_sh.at[src_v.at[c]], rows, sem).wait()
                nrows, nsem = bufs[(b + NBUF - 1) % NBUF]

                @pl.when(c + NBUF - 1 < PCH)
                def _():
                    pltpu.make_async_copy(y_sh.at[src_v.at[c + NBUF - 1]],
                                          nrows, nsem).start()

                pltpu.sync_copy(rows, acc_sh.at[dst_v.at[c]], add=True)
            return carry

        lax.fori_loop(0, PCH // NBUF, _edge_group, 0)
    plsc.subcore_barrier()

    @pl.when(cid == 0)
    def _():
        pltpu.sync_copy(acc_sh.at[pl.ds(row0, ROWS_PER_TILE)],
                        out_lo.at[pl.ds(row0, ROWS_PER_TILE)])

    @pl.when(cid == 1)
    def _():
        pltpu.sync_copy(acc_sh.at[pl.ds(row0, ROWS_PER_TILE)],
                        out_hi.at[pl.ds(row0, ROWS_PER_TILE)])


def _tc_pre_body(degT_ref, x_ref, w_ref, ylo_ref, yhi_ref, dis_ref):
    deg = 1.0 + jnp.sum(degT_ref[...], axis=1, keepdims=True)
    dis = lax.rsqrt(deg)
    y = dis * jnp.dot(x_ref[...], w_ref[...],
                      preferred_element_type=jnp.float32)
    ylo_ref[...] = y[:, :HD]
    yhi_ref[...] = y[:, HD:]
    dis_ref[...] = dis


def _halves(slo_ref, shi_ref, ylo_ref, yhi_ref, dis_ref, b_ref):
    dis = dis_ref[...]
    b = b_ref[...]
    h_lo = jnp.maximum(dis * (slo_ref[...] + ylo_ref[...]) + b[:, :HD], 0.0)
    h_hi = jnp.maximum(dis * (shi_ref[...] + yhi_ref[...]) + b[:, HD:], 0.0)
    return h_lo, h_hi


def _tc_mid_body(slo_ref, shi_ref, ylo_ref, yhi_ref, dis_ref, b_ref, w_ref,
                 olo_ref, ohi_ref):
    h_lo, h_hi = _halves(slo_ref, shi_ref, ylo_ref, yhi_ref, dis_ref, b_ref)
    w = w_ref[...]
    y = dis_ref[...] * (
        jnp.dot(h_lo, w[:HD, :], preferred_element_type=jnp.float32)
        + jnp.dot(h_hi, w[HD:, :], preferred_element_type=jnp.float32))
    olo_ref[...] = y[:, :HD]
    ohi_ref[...] = y[:, HD:]


def _tc_post_body(slo_ref, shi_ref, ylo_ref, yhi_ref, dis_ref, b_ref, bt_ref,
                  f1w_ref, f1b_ref, f2w_ref, f2b_ref, cw_ref, cb_ref, o_ref,
                  psum, cnt):
    i = pl.program_id(0)

    @pl.when(i == 0)
    def _():
        psum[...] = jnp.zeros((G, D), jnp.float32)
        cnt[...] = jnp.zeros((G, 1), jnp.float32)

    h_lo, h_hi = _halves(slo_ref, shi_ref, ylo_ref, yhi_ref, dis_ref, b_ref)
    gids = lax.broadcasted_iota(jnp.int32, (RB, G), 1)
    onehot = (bt_ref[...] == gids).astype(jnp.float32)
    dn = (((0,), (0,)), ((), ()))
    psum[:, :HD] += lax.dot_general(onehot, h_lo, dn,
                                    preferred_element_type=jnp.float32)
    psum[:, HD:] += lax.dot_general(onehot, h_hi, dn,
                                    preferred_element_type=jnp.float32)
    cnt[...] += lax.dot_general(onehot, jnp.ones((RB, 1), jnp.float32), dn,
                                preferred_element_type=jnp.float32)

    @pl.when(i == TC_GRID - 1)
    def _():
        pooled = psum[...] / jnp.maximum(cnt[...], 1.0)
        z = jnp.maximum(
            jnp.dot(pooled, f1w_ref[...], preferred_element_type=jnp.float32)
            + f1b_ref[...], 0.0)
        z = jnp.maximum(
            jnp.dot(z, f2w_ref[...], preferred_element_type=jnp.float32)
            + f2b_ref[...], 0.0)
        o_ref[...] = (jnp.dot(z, cw_ref[...], preferred_element_type=jnp.float32)
                      + cb_ref[...])


def _row_spec(cols):
    return pl.BlockSpec((RB, cols), lambda i: (i, 0))


def _whole_spec(rows, cols):
    return pl.BlockSpec((rows, cols), lambda i: (0, 0))


_tc_pre = pl.pallas_call(
    _tc_pre_body,
    grid=(TC_GRID,),
    in_specs=[_row_spec(NUM_WORKERS), _row_spec(D), _whole_spec(D, D)],
    out_specs=[_row_spec(HD), _row_spec(HD), _row_spec(1)],
    out_shape=[
        jax.ShapeDtypeStruct((NP, HD), jnp.float32),
        jax.ShapeDtypeStruct((NP, HD), jnp.float32),
        jax.ShapeDtypeStruct((NP, 1), jnp.float32),
    ],
)

_tc_mid = pl.pallas_call(
    _tc_mid_body,
    grid=(TC_GRID,),
    in_specs=[_row_spec(HD), _row_spec(HD), _row_spec(HD), _row_spec(HD),
              _row_spec(1), _whole_spec(1, D), _whole_spec(D, D)],
    out_specs=[_row_spec(HD), _row_spec(HD)],
    out_shape=[
        jax.ShapeDtypeStruct((NP, HD), jnp.float32),
        jax.ShapeDtypeStruct((NP, HD), jnp.float32),
    ],
)

_tc_post = pl.pallas_call(
    _tc_post_body,
    grid=(TC_GRID,),
    in_specs=[_row_spec(HD), _row_spec(HD), _row_spec(HD), _row_spec(HD),
              _row_spec(1), _whole_spec(1, D), _row_spec(1),
              _whole_spec(D, D), _whole_spec(1, D),
              _whole_spec(D, D), _whole_spec(1, D),
              _whole_spec(D, C), _whole_spec(1, C)],
    out_specs=_whole_spec(G, C),
    out_shape=jax.ShapeDtypeStruct((G, C), jnp.float32),
    scratch_shapes=[
        pltpu.VMEM((G, D), jnp.float32),
        pltpu.VMEM((G, 1), jnp.float32),
    ],
)


def kernel(x, edge_index, batch, W1, b1, W2, b2, fc1_W, fc1_b, fc2_W, fc2_b,
           cls_W, cls_b):
    src2d = jnp.pad(edge_index[0], (0, EP - E)).reshape(EP // EW, EW)
    dst2d = jnp.pad(edge_index[1], (0, EP - E),
                    constant_values=TRASH).reshape(EP // EW, EW)
    x_p = jnp.pad(x, ((0, NP - N), (0, 0)))
    batch_p = jnp.pad(batch, (0, NP - N), constant_values=G).reshape(NP, 1)

    sc_degree = _sc_degree_kernel()
    sc_scatter = _sc_scatter_kernel()
    deg_p = sc_degree(dst2d).reshape(NUM_WORKERS, NP)
    y1lo, y1hi, dis = _tc_pre(deg_p.T, x_p, W1)
    s1lo, s1hi = sc_scatter(src2d, dst2d, y1lo, y1hi)
    y2lo, y2hi = _tc_mid(s1lo, s1hi, y1lo, y1hi, dis, b1.reshape(1, D), W2)
    s2lo, s2hi = sc_scatter(src2d, dst2d, y2lo, y2hi)
    out = _tc_post(s2lo, s2hi, y2lo, y2hi, dis, b2.reshape(1, D), batch_p,
                   fc1_W, fc1_b.reshape(1, D), fc2_W, fc2_b.reshape(1, D),
                   cls_W, cls_b.reshape(1, C))
    return out


# speedup vs baseline: 1.0478x; 1.0478x over previous
import functools

import jax
import jax.numpy as jnp
from jax import lax
from jax.experimental import pallas as pl
from jax.experimental.pallas import tpu as pltpu
from jax.experimental.pallas import tpu_sc as plsc

N = 10000
NP = 10240
E = 320000
D = 128
HD = D // 2
G = 128
C = 10

NUM_WORKERS = 32
EW = 64
EP = 327680
NBUF = 4
TRASH = NP - 1
ROWS_PER_TILE = NP // 16
TCH = EP // (16 * EW)
PARTS = 8
PCH = TCH // PARTS
DCH = EP // (NUM_WORKERS * EW)
RB = 2048
TC_GRID = NP // RB

@functools.cache
def _sc_degree_kernel():
    mesh = plsc.VectorSubcoreMesh(core_axis_name="c", subcore_axis_name="s")
    return functools.partial(
        pl.kernel,
        mesh=mesh,
        out_type=jax.ShapeDtypeStruct((NUM_WORKERS * NP,), jnp.float32),
        scratch_types=[
            pltpu.VMEM((DCH, EW), jnp.int32),
            pltpu.VMEM((NP,), jnp.float32),
        ],
        compiler_params=pltpu.CompilerParams(needs_layout_passes=False),
    )(_sc_degree_body)


def _sc_degree_body(dst_hbm, out_hbm, dst_v, deg_v):
    cid = lax.axis_index("c")
    sid = lax.axis_index("s")
    w = cid * 16 + sid
    pltpu.sync_copy(dst_hbm.at[pl.ds(w * DCH, DCH)], dst_v)

    zeros16 = jnp.zeros((16,), jnp.float32)

    def _zero(i, carry):
        deg_v[pl.ds(i * 16, 16)] = zeros16
        return carry

    lax.fori_loop(0, NP // 16, _zero, 0)

    ones16 = jnp.ones((16,), jnp.float32)

    def _count(j, carry):
        for k in range(EW // 16):
            idx = dst_v[j, pl.ds(k * 16, 16)]
            plsc.addupdate_scatter(deg_v, [idx], ones16)
        return carry

    lax.fori_loop(0, DCH, _count, 0)
    pltpu.sync_copy(deg_v, out_hbm.at[pl.ds(w * NP, NP)])


@functools.cache
def _sc_scatter_kernel():
    mesh = plsc.VectorSubcoreMesh(core_axis_name="c", subcore_axis_name="s")
    return functools.partial(
        pl.kernel,
        mesh=mesh,
        out_type=(
            jax.ShapeDtypeStruct((NP, HD), jnp.float32),
            jax.ShapeDtypeStruct((NP, HD), jnp.float32),
        ),
        scratch_types=[
            pltpu.VMEM((PCH, EW), jnp.int32),
            pltpu.VMEM((PCH, EW), jnp.int32),
        ]
        + [pltpu.VMEM((EW, HD), jnp.float32) for _ in range(NBUF)]
        + [
            pltpu.VMEM_SHARED((NP, HD), jnp.float32),
            pltpu.VMEM_SHARED((NP, HD), jnp.float32),
        ]
        + [pltpu.SemaphoreType.DMA for _ in range(NBUF)],
        compiler_params=pltpu.CompilerParams(needs_layout_passes=False,
                                             use_tc_tiling_on_sc=False),
    )(_sc_scatter_body)


def _sc_scatter_body(src_hbm, dst_hbm, ylo_hbm, yhi_hbm, out_lo, out_hi,
                     src_v, dst_v, *rest):
    rows_bufs = rest[:NBUF]
    y_sh = rest[NBUF]
    acc_sh = rest[NBUF + 1]
    sems = rest[NBUF + 2:]
    rows0 = rows_bufs[0]
    cid = lax.axis_index("c")
    sid = lax.axis_index("s")
    row0 = sid * ROWS_PER_TILE

    @pl.when(cid == 0)
    def _():
        pltpu.sync_copy(ylo_hbm.at[pl.ds(row0, ROWS_PER_TILE)],
                        y_sh.at[pl.ds(row0, ROWS_PER_TILE)])

    @pl.when(cid == 1)
    def _():
        pltpu.sync_copy(yhi_hbm.at[pl.ds(row0, ROWS_PER_TILE)],
                        y_sh.at[pl.ds(row0, ROWS_PER_TILE)])

    zeros16 = jnp.zeros((16,), jnp.float32)

    def _zero(i, carry):
        for k in range(HD // 16):
            rows0[i, pl.ds(k * 16, 16)] = zeros16
        return carry

    lax.fori_loop(0, EW, _zero, 0)

    def _zero_acc(k, carry):
        pltpu.sync_copy(rows0, acc_sh.at[pl.ds(row0 + k * EW, EW)])
        return carry

    lax.fori_loop(0, ROWS_PER_TILE // EW, _zero_acc, 0)
    plsc.subcore_barrier()

    bufs = tuple(zip(rows_bufs, sems))
    for h in range(PARTS):
        base = sid * TCH + h * PCH
        pltpu.sync_copy(src_hbm.at[pl.ds(base, PCH)], src_v)
        pltpu.sync_copy(dst_hbm.at[pl.ds(base, PCH)], dst_v)
        for p in range(NBUF - 1):
            pltpu.make_async_copy(y_sh.at[src_v.at[p]], rows_bufs[p],
                                  sems[p]).start()

        def _edge_group(jj, carry):
            for b, (rows, sem) in enumerate(bufs):
                c = jj * NBUF + b
                pltpu.make_async_copy(y_sh.at[src_v.at[c]], rows, sem).wait()
                nrows, nsem = bufs[(b + NBUF - 1) % NBUF]

                @pl.when(c + NBUF - 1 < PCH)
                def _():
                    pltpu.make_async_copy(y_sh.at[src_v.at[c + NBUF - 1]],
                                          nrows, nsem).start()

                pltpu.sync_copy(rows, acc_sh.at[dst_v.at[c]], add=True)
            return carry

        lax.fori_loop(0, PCH // NBUF, _edge_group, 0)
    plsc.subcore_barrier()

    @pl.when(cid == 0)
    def _():
        pltpu.sync_copy(acc_sh.at[pl.ds(row0, ROWS_PER_TILE)],
                        out_lo.at[pl.ds(row0, ROWS_PER_TILE)])

    @pl.when(cid == 1)
    def _():
        pltpu.sync_copy(acc_sh.at[pl.ds(row0, ROWS_PER_TILE)],
                        out_hi.at[pl.ds(row0, ROWS_PER_TILE)])


def _tc_pre_body(degT_ref, x_ref, w_ref, ylo_ref, yhi_ref, dis_ref):
    deg = 1.0 + jnp.sum(degT_ref[...], axis=1, keepdims=True)
    dis = lax.rsqrt(deg)
    y = dis * jnp.dot(x_ref[...], w_ref[...],
                      preferred_element_type=jnp.float32)
    ylo_ref[...] = y[:, :HD]
    yhi_ref[...] = y[:, HD:]
    dis_ref[...] = dis


def _halves(slo_ref, shi_ref, ylo_ref, yhi_ref, dis_ref, b_ref):
    dis = dis_ref[...]
    b = b_ref[...]
    h_lo = jnp.maximum(dis * (slo_ref[...] + ylo_ref[...]) + b[:, :HD], 0.0)
    h_hi = jnp.maximum(dis * (shi_ref[...] + yhi_ref[...]) + b[:, HD:], 0.0)
    return h_lo, h_hi


def _tc_mid_body(slo_ref, shi_ref, ylo_ref, yhi_ref, dis_ref, b_ref, w_ref,
                 olo_ref, ohi_ref):
    h_lo, h_hi = _halves(slo_ref, shi_ref, ylo_ref, yhi_ref, dis_ref, b_ref)
    w = w_ref[...]
    y = dis_ref[...] * (
        jnp.dot(h_lo, w[:HD, :], preferred_element_type=jnp.float32)
        + jnp.dot(h_hi, w[HD:, :], preferred_element_type=jnp.float32))
    olo_ref[...] = y[:, :HD]
    ohi_ref[...] = y[:, HD:]


def _tc_post_body(slo_ref, shi_ref, ylo_ref, yhi_ref, dis_ref, b_ref, bt_ref,
                  f1w_ref, f1b_ref, f2w_ref, f2b_ref, cw_ref, cb_ref, o_ref,
                  psum, cnt):
    i = pl.program_id(0)

    @pl.when(i == 0)
    def _():
        psum[...] = jnp.zeros((G, D), jnp.float32)
        cnt[...] = jnp.zeros((G, 1), jnp.float32)

    h_lo, h_hi = _halves(slo_ref, shi_ref, ylo_ref, yhi_ref, dis_ref, b_ref)
    gids = lax.broadcasted_iota(jnp.int32, (RB, G), 1)
    onehot = (bt_ref[...] == gids).astype(jnp.float32)
    dn = (((0,), (0,)), ((), ()))
    psum[:, :HD] += lax.dot_general(onehot, h_lo, dn,
                                    preferred_element_type=jnp.float32)
    psum[:, HD:] += lax.dot_general(onehot, h_hi, dn,
                                    preferred_element_type=jnp.float32)
    cnt[...] += lax.dot_general(onehot, jnp.ones((RB, 1), jnp.float32), dn,
                                preferred_element_type=jnp.float32)

    @pl.when(i == TC_GRID - 1)
    def _():
        pooled = psum[...] / jnp.maximum(cnt[...], 1.0)
        z = jnp.maximum(
            jnp.dot(pooled, f1w_ref[...], preferred_element_type=jnp.float32)
            + f1b_ref[...], 0.0)
        z = jnp.maximum(
            jnp.dot(z, f2w_ref[...], preferred_element_type=jnp.float32)
            + f2b_ref[...], 0.0)
        o_ref[...] = (jnp.dot(z, cw_ref[...], preferred_element_type=jnp.float32)
                      + cb_ref[...])


def _row_spec(cols):
    return pl.BlockSpec((RB, cols), lambda i: (i, 0))


def _whole_spec(rows, cols):
    return pl.BlockSpec((rows, cols), lambda i: (0, 0))


_tc_pre = pl.pallas_call(
    _tc_pre_body,
    grid=(TC_GRID,),
    in_specs=[_row_spec(NUM_WORKERS), _row_spec(D), _whole_spec(D, D)],
    out_specs=[_row_spec(HD), _row_spec(HD), _row_spec(1)],
    out_shape=[
        jax.ShapeDtypeStruct((NP, HD), jnp.float32),
        jax.ShapeDtypeStruct((NP, HD), jnp.float32),
        jax.ShapeDtypeStruct((NP, 1), jnp.float32),
    ],
)

_tc_mid = pl.pallas_call(
    _tc_mid_body,
    grid=(TC_GRID,),
    in_specs=[_row_spec(HD), _row_spec(HD), _row_spec(HD), _row_spec(HD),
              _row_spec(1), _whole_spec(1, D), _whole_spec(D, D)],
    out_specs=[_row_spec(HD), _row_spec(HD)],
    out_shape=[
        jax.ShapeDtypeStruct((NP, HD), jnp.float32),
        jax.ShapeDtypeStruct((NP, HD), jnp.float32),
    ],
)

_tc_post = pl.pallas_call(
    _tc_post_body,
    grid=(TC_GRID,),
    in_specs=[_row_spec(HD), _row_spec(HD), _row_spec(HD), _row_spec(HD),
              _row_spec(1), _whole_spec(1, D), _row_spec(1),
              _whole_spec(D, D), _whole_spec(1, D),
              _whole_spec(D, D), _whole_spec(1, D),
              _whole_spec(D, C), _whole_spec(1, C)],
    out_specs=_whole_spec(G, C),
    out_shape=jax.ShapeDtypeStruct((G, C), jnp.float32),
    scratch_shapes=[
        pltpu.VMEM((G, D), jnp.float32),
        pltpu.VMEM((G, 1), jnp.float32),
    ],
)


def kernel(x, edge_index, batch, W1, b1, W2, b2, fc1_W, fc1_b, fc2_W, fc2_b,
           cls_W, cls_b):
    src2d = jnp.pad(edge_index[0], (0, EP - E)).reshape(EP // EW, EW)
    dst2d = jnp.pad(edge_index[1], (0, EP - E),
                    constant_values=TRASH).reshape(EP // EW, EW)
    x_p = jnp.pad(x, ((0, NP - N), (0, 0)))
    batch_p = jnp.pad(batch, (0, NP - N), constant_values=G).reshape(NP, 1)

    sc_degree = _sc_degree_kernel()
    sc_scatter = _sc_scatter_kernel()
    deg_p = sc_degree(dst2d).reshape(NUM_WORKERS, NP)
    y1lo, y1hi, dis = _tc_pre(deg_p.T, x_p, W1)
    s1lo, s1hi = sc_scatter(src2d, dst2d, y1lo, y1hi)
    y2lo, y2hi = _tc_mid(s1lo, s1hi, y1lo, y1hi, dis, b1.reshape(1, D), W2)
    s2lo, s2hi = sc_scatter(src2d, dst2d, y2lo, y2hi)
    out = _tc_post(s2lo, s2hi, y2lo, y2hi, dis, b2.reshape(1, D), batch_p,
                   fc1_W, fc1_b.reshape(1, D), fc2_W, fc2_b.reshape(1, D),
                   cls_W, cls_b.reshape(1, C))
    return out
